# baseline (speedup 1.0000x reference)
import jax
import jax.numpy as jnp
from jax.experimental import pallas as pl
from jax.experimental.pallas import tpu as pltpu

_FEAT = 128
_PIDS = 100000
_CQ = 5000
_SCALAR = 30.0
_B = 128

_TILE = 5000
_T_LUT = _PIDS // _TILE
_NS = 5
_SPAN = _T_LUT // _NS
_GRID = _SPAN + 1
_IGNORE = 5554
_LN2 = 0.6931471805599453


def _oim_body(m2_ref, x_ref, lbl_ref, c0_ref, c1_ref, c2_ref, c3_ref, c4_ref,
              w0_ref, w1_ref, w2_ref, w3_ref, w4_ref, cq_ref, out_ref, s_s, t_s):
    i = pl.program_id(0)

    @pl.when(i == 0)
    def _init():
        s_s[...] = jnp.zeros((_B, 1), jnp.float32)
        t_s[...] = jnp.zeros((_B, 1), jnp.float32)

    x = x_ref[...]
    m2 = m2_ref[0]

    def _accumulate(w, c, base, with_target):
        s2 = jax.lax.dot_general(
            x, w.astype(jnp.bfloat16), (((1,), (1,)), ((), ())),
            preferred_element_type=jnp.float32)
        s2 = s2 * c[None, :]
        p = jnp.exp2(s2 - m2)
        s_s[...] += jnp.sum(p, axis=1, keepdims=True)
        if with_target:
            col = lbl_ref[...] - base
            iota = jax.lax.broadcasted_iota(jnp.int32, (_B, _TILE), 1)
            hit = jnp.where(iota == col, s2, 0.0)
            t_s[...] += jnp.sum(hit, axis=1, keepdims=True)

    @pl.when(i < _SPAN)
    def _lut_phase():
        _accumulate(w0_ref[...], c0_ref[0, 0, :], i * _TILE, True)
        _accumulate(w1_ref[...], c1_ref[0, 0, :], (i + _SPAN) * _TILE, True)
        _accumulate(w2_ref[...], c2_ref[0, 0, :], (i + 2 * _SPAN) * _TILE, True)
        _accumulate(w3_ref[...], c3_ref[0, 0, :], (i + 3 * _SPAN) * _TILE, True)
        _accumulate(w4_ref[...], c4_ref[0, 0, :], (i + 4 * _SPAN) * _TILE, True)

    @pl.when(i == _SPAN)
    def _cq_phase():
        _accumulate(cq_ref[...], c0_ref[0, 0, :], _PIDS, False)

    @pl.when(i == _GRID - 1)
    def _finish():
        lse = m2 * _LN2 + jnp.log(s_s[...])
        nll = lse - t_s[...] * _LN2
        valid = (lbl_ref[...] != _IGNORE).astype(jnp.float32)
        denom = jnp.maximum(jnp.sum(valid), 1.0)
        out_ref[...] = (jnp.sum(nll * valid) / denom).reshape(1, 1)


def _c_spec(k):
    return pl.BlockSpec(
        (1, 1, _TILE),
        lambda i, k=k: (jnp.where(i < _SPAN, i + k * _SPAN, _T_LUT), 0, 0))


def _w_spec(k):
    return pl.BlockSpec(
        (_TILE, _FEAT),
        lambda i, k=k: (jnp.minimum(i, _SPAN - 1) + k * _SPAN, 0))


def kernel(inputs, roi_label, roi_ious, lut, cq, reliability):
    del roi_ious
    lbl = roi_label.reshape(_B, 1).astype(jnp.int32) - 1
    inputs = inputs.astype(jnp.bfloat16)
    c = reliability * (_SCALAR * 1.4426950408889634)
    m2 = jnp.max(jnp.abs(c)).reshape(1)
    c3 = c.reshape(_T_LUT + 1, 1, _TILE)
    out = pl.pallas_call(
        _oim_body,
        grid=(_GRID,),
        in_specs=[
            pl.BlockSpec(memory_space=pltpu.SMEM),
            pl.BlockSpec((_B, _FEAT), lambda i: (0, 0)),
            pl.BlockSpec((_B, 1), lambda i: (0, 0)),
            _c_spec(0), _c_spec(1), _c_spec(2), _c_spec(3), _c_spec(4),
            _w_spec(0), _w_spec(1), _w_spec(2), _w_spec(3), _w_spec(4),
            pl.BlockSpec((_CQ, _FEAT), lambda i: (0, 0)),
        ],
        out_specs=pl.BlockSpec((1, 1), lambda i: (0, 0)),
        out_shape=jax.ShapeDtypeStruct((1, 1), jnp.float32),
        scratch_shapes=[
            pltpu.VMEM((_B, 1), jnp.float32),
            pltpu.VMEM((_B, 1), jnp.float32),
        ],
    )(m2, inputs, lbl, c3, c3, c3, c3, c3, lut, lut, lut, lut, lut, cq)
    return out[0, 0]
